# Initial kernel scaffold; baseline (speedup 1.0000x reference)
import jax
import jax.numpy as jnp
from jax.experimental import pallas as pl


def kernel(hidden_states, gate_w, gate_up, down):
    raise NotImplementedError("write your pallas kernel here")


# speedup vs baseline: 1.5186x; 1.5186x over previous
import functools

import jax
import jax.numpy as jnp
from jax import lax
from jax.experimental import pallas as pl
from jax.experimental.pallas import tpu as pltpu
from jax.experimental.pallas import tpu_sc as plsc

E = 8
D = 2048
F = 1024
T = 8192
BT = 256
GMAX = (T * 2 + E * (BT - 1) + BT - 1) // BT
PMAX = GMAX * BT
CHUNK = 128
NCHUNK = T // CHUNK

NSC = 2
NW = NSC * 16
TPW = T // NW


def _router_body(x_ref, gwt_ref, oh1_ref, oh2_ref, w1_ref, w2_ref):
    x = x_ref[...]
    logits = jnp.dot(x, gwt_ref[...], preferred_element_type=jnp.float32)
    iota = lax.broadcasted_iota(jnp.int32, logits.shape, 1)
    m1 = jnp.max(logits, axis=1, keepdims=True)
    idx1 = jnp.min(jnp.where(logits == m1, iota, E), axis=1, keepdims=True)
    oh1 = iota == idx1
    l2 = jnp.where(oh1, -jnp.inf, logits)
    m2 = jnp.max(l2, axis=1, keepdims=True)
    idx2 = jnp.min(jnp.where(l2 == m2, iota, E), axis=1, keepdims=True)
    oh2 = iota == idx2
    em = jnp.exp(m2 - m1)
    denom = 1.0 + em
    oh1_ref[...] = oh1.astype(jnp.float32)
    oh2_ref[...] = oh2.astype(jnp.float32)
    w1_ref[...] = 1.0 / denom
    w2_ref[...] = em / denom


def _router(x, gwt):
    bt = 1024
    return pl.pallas_call(
        _router_body,
        grid=(T // bt,),
        in_specs=[
            pl.BlockSpec((bt, D), lambda i: (i, 0)),
            pl.BlockSpec((D, E), lambda i: (0, 0)),
        ],
        out_specs=[
            pl.BlockSpec((bt, E), lambda i: (i, 0)),
            pl.BlockSpec((bt, E), lambda i: (i, 0)),
            pl.BlockSpec((bt, 1), lambda i: (i, 0)),
            pl.BlockSpec((bt, 1), lambda i: (i, 0)),
        ],
        out_shape=[
            jax.ShapeDtypeStruct((T, E), jnp.float32),
            jax.ShapeDtypeStruct((T, E), jnp.float32),
            jax.ShapeDtypeStruct((T, 1), jnp.float32),
            jax.ShapeDtypeStruct((T, 1), jnp.float32),
        ],
    )(x, gwt)


def _dispatch_body(oh1_ref, oh2_ref, qa_ref, qb_ref, eot_ref, rank_ref):
    r_io = lax.broadcasted_iota(jnp.int32, (CHUNK, CHUNK), 0)
    c_io = lax.broadcasted_iota(jnp.int32, (CHUNK, CHUNK), 1)
    tril = (c_io <= r_io).astype(jnp.float32)

    def body(i, carry):
        m = (oh1_ref[pl.ds(i * CHUNK, CHUNK), :]
             + oh2_ref[pl.ds(i * CHUNK, CHUNK), :])
        incl = jnp.dot(tril, m, preferred_element_type=jnp.float32,
                       precision=lax.Precision.HIGHEST)
        rank_ref[pl.ds(i * CHUNK, CHUNK), :] = incl - m + carry
        return carry + incl[CHUNK - 1:CHUNK, :]

    counts = lax.fori_loop(0, NCHUNK, body, jnp.zeros((1, E), jnp.float32))
    padded = jnp.ceil(counts * (1.0 / BT)) * BT
    r8 = lax.broadcasted_iota(jnp.int32, (E, E), 0)
    c8 = lax.broadcasted_iota(jnp.int32, (E, E), 1)
    upper = (r8 <= c8).astype(jnp.float32)
    start = jnp.dot(padded, upper, preferred_element_type=jnp.float32,
                    precision=lax.Precision.HIGHEST) - padded

    ts = start * (1.0 / BT)
    g_io = lax.broadcasted_iota(jnp.int32, (GMAX, E), 0).astype(jnp.float32)
    cmp = (jnp.broadcast_to(ts, (GMAX, E)) <= g_io).astype(jnp.float32)
    eot_ref[...] = (jnp.sum(cmp, axis=1, keepdims=True) - 1.0).astype(jnp.int32)

    pos = rank_ref[...] + jnp.broadcast_to(start, (T, E))
    qa = jnp.sum(oh1_ref[...] * pos, axis=1, keepdims=True)
    qb = jnp.sum(oh2_ref[...] * pos, axis=1, keepdims=True)
    qa_ref[...] = qa.astype(jnp.int32)
    qb_ref[...] = qb.astype(jnp.int32)


def _dispatch(oh1, oh2):
    return pl.pallas_call(
        _dispatch_body,
        out_shape=[
            jax.ShapeDtypeStruct((T, 1), jnp.int32),
            jax.ShapeDtypeStruct((T, 1), jnp.int32),
            jax.ShapeDtypeStruct((GMAX, 1), jnp.int32),
        ],
        scratch_shapes=[pltpu.VMEM((T, E), jnp.float32)],
    )(oh1, oh2)


_RB3 = 32


def _scatter_x(x, qa, qb):
    mesh = plsc.VectorSubcoreMesh(core_axis_name="c", subcore_axis_name="s")

    @functools.partial(
        pl.kernel,
        out_type=jax.ShapeDtypeStruct((PMAX, D), jnp.float32),
        mesh=mesh,
        scratch_types=[
            pltpu.VMEM((_RB3, D), jnp.float32),
            pltpu.VMEM((_RB3,), jnp.int32),
            pltpu.VMEM((_RB3,), jnp.int32),
            pltpu.SemaphoreType.DMA,
        ],
    )
    def k(x_hbm, qa_hbm, qb_hbm, xg_hbm, xbuf, ia, ib, sem):
        wid = lax.axis_index("s") * NSC + lax.axis_index("c")
        base = wid * TPW

        def body(j, _):
            rb = base + j * _RB3
            pltpu.sync_copy(x_hbm.at[pl.ds(rb, _RB3), :], xbuf)
            pltpu.sync_copy(qa_hbm.at[pl.ds(rb, _RB3)], ia)
            pltpu.sync_copy(qb_hbm.at[pl.ds(rb, _RB3)], ib)
            pltpu.async_copy(xbuf, xg_hbm.at[ia], sem).wait()
            pltpu.async_copy(xbuf, xg_hbm.at[ib], sem).wait()
            return 0

        lax.fori_loop(0, TPW // _RB3, body, 0)

    return k(x, qa, qb)


def _gmm_body(eot_ref, xg_ref, gu_ref, dw_ref, out_ref):
    xb = xg_ref[...].astype(jnp.bfloat16)
    w1 = gu_ref[0]
    gu = lax.dot_general(xb, w1, (((1,), (1,)), ((), ())),
                         preferred_element_type=jnp.float32)
    g = gu[:, :F]
    u = gu[:, F:]
    h = (g * lax.logistic(g) * u).astype(jnp.bfloat16)
    w2 = dw_ref[0]
    out_ref[...] = lax.dot_general(h, w2, (((1,), (1,)), ((), ())),
                                   preferred_element_type=jnp.float32)


def _gmm(eot, xg, gub, dwb):
    grid_spec = pltpu.PrefetchScalarGridSpec(
        num_scalar_prefetch=1,
        grid=(GMAX,),
        in_specs=[
            pl.BlockSpec((BT, D), lambda g, eot: (g, 0)),
            pl.BlockSpec((1, 2 * F, D), lambda g, eot: (eot[g], 0, 0)),
            pl.BlockSpec((1, D, F), lambda g, eot: (eot[g], 0, 0)),
        ],
        out_specs=pl.BlockSpec((BT, D), lambda g, eot: (g, 0)),
    )
    return pl.pallas_call(
        _gmm_body,
        grid_spec=grid_spec,
        out_shape=jax.ShapeDtypeStruct((PMAX, D), jnp.float32),
        compiler_params=pltpu.CompilerParams(
            dimension_semantics=("arbitrary",),
            vmem_limit_bytes=100 * 1024 * 1024),
    )(eot, xg, gub, dwb)


_RB5 = 16


def _combine(hg, qa, qb, w1, w2):
    mesh = plsc.VectorSubcoreMesh(core_axis_name="c", subcore_axis_name="s")

    @functools.partial(
        pl.kernel,
        out_type=jax.ShapeDtypeStruct((T, D), jnp.float32),
        mesh=mesh,
        scratch_types=[
            pltpu.VMEM((_RB5, D), jnp.float32),
            pltpu.VMEM((_RB5, D), jnp.float32),
            pltpu.VMEM((_RB5,), jnp.int32),
            pltpu.VMEM((_RB5,), jnp.int32),
            pltpu.VMEM((_RB5,), jnp.float32),
            pltpu.VMEM((_RB5,), jnp.float32),
            pltpu.SemaphoreType.DMA,
        ],
    )
    def k(hg_hbm, qa_hbm, qb_hbm, w1_hbm, w2_hbm, out_hbm,
          buf_a, buf_b, ia, ib, wa, wb, sem):
        wid = lax.axis_index("s") * NSC + lax.axis_index("c")
        base = wid * TPW

        def chunk(j, _):
            rb = base + j * _RB5
            pltpu.sync_copy(qa_hbm.at[pl.ds(rb, _RB5)], ia)
            pltpu.sync_copy(qb_hbm.at[pl.ds(rb, _RB5)], ib)
            pltpu.sync_copy(w1_hbm.at[pl.ds(rb, _RB5)], wa)
            pltpu.sync_copy(w2_hbm.at[pl.ds(rb, _RB5)], wb)
            pltpu.async_copy(hg_hbm.at[ia], buf_a, sem).wait()
            pltpu.async_copy(hg_hbm.at[ib], buf_b, sem).wait()
            wav = wa[...]
            wbv = wb[...]
            for r in range(_RB5):
                sa = wav[r]
                sb = wbv[r]

                def col(c, _):
                    a = buf_a[r, pl.ds(c * 16, 16)]
                    b = buf_b[r, pl.ds(c * 16, 16)]
                    buf_a[r, pl.ds(c * 16, 16)] = a * sa + b * sb
                    return 0

                lax.fori_loop(0, D // 16, col, 0)
            pltpu.sync_copy(buf_a, out_hbm.at[pl.ds(rb, _RB5), :])
            return 0

        lax.fori_loop(0, TPW // _RB5, chunk, 0)

    return k(hg, qa, qb, w1, w2)


def kernel(hidden_states, gate_w, gate_up, down):
    b, s, d = hidden_states.shape
    x = hidden_states.reshape(-1, d)
    oh1, oh2, w1, w2 = _router(x, gate_w.T)
    qa2, qb2, eot2 = _dispatch(oh1, oh2)
    qa = qa2.reshape(T)
    qb = qb2.reshape(T)
    eot = eot2.reshape(GMAX)
    gub = gate_up.astype(jnp.bfloat16)
    dwb = down.astype(jnp.bfloat16)
    xg = _scatter_x(x, qa, qb)
    hg = _gmm(eot, xg, gub, dwb)
    out = _combine(hg, qa, qb, w1.reshape(T), w2.reshape(T))
    return out.reshape(b, s, d)
